# Initial kernel scaffold; baseline (speedup 1.0000x reference)
import jax
import jax.numpy as jnp
from jax.experimental import pallas as pl


def kernel(x, t, ada_w, ada_b, ada_ln_w, ada_ln_b, Wqkv, qn_w, qn_b, kn_w, kn_b, Wout, mlp_ln_w, mlp_ln_b, Wg, fc1s, fc2s, b1s, b2s):
    raise NotImplementedError("write your pallas kernel here")


# speedup vs baseline: 1.3175x; 1.3175x over previous
import functools

import jax
import jax.numpy as jnp
from jax import lax
from jax.experimental import pallas as pl
from jax.experimental.pallas import tpu as pltpu
from jax.experimental.pallas import tpu_sc as plsc

F32 = jnp.float32
BF16 = jnp.bfloat16
EPS = 1e-5
_INTERPRET = False


def _k0_body(t_ref, w_ref, b_ref, o_ref):
    t = t_ref[...]
    s = t * jax.nn.sigmoid(t)
    o_ref[...] = (
        lax.dot_general(s, w_ref[...], (((1,), (1,)), ((), ())),
                        preferred_element_type=F32)
        + b_ref[...]
    )


def _k0(t2, ada_w, ada_b):
    return pl.pallas_call(
        _k0_body,
        out_shape=jax.ShapeDtypeStruct(t2.shape, F32),
        interpret=_INTERPRET,
    )(t2, ada_w, ada_b)


def _k1_body(x_ref, sh_ref, lnw_ref, lnb_ref, wqkv_ref,
             qnw_ref, qnb_ref, knw_ref, knb_ref, o_ref, *, d, h, dh, spb):
    x = x_ref[...]
    mu = jnp.mean(x, axis=1, keepdims=True)
    xc = x - mu
    var = jnp.mean(xc * xc, axis=1, keepdims=True)
    sh = sh_ref[pl.ds(pl.program_id(0) // spb, 1), :]
    hpre = xc * lax.rsqrt(var + EPS) * lnw_ref[...] + lnb_ref[...] + sh
    qkv = lax.dot_general(hpre, wqkv_ref[...], (((1,), (1,)), ((), ())),
                          preferred_element_type=F32)
    q = qkv[:, :d]
    k = qkv[:, d:2 * d]
    v = qkv[:, 2 * d:]
    r = lax.broadcasted_iota(jnp.int32, (d, h), 0)
    c = lax.broadcasted_iota(jnp.int32, (d, h), 1)
    seg = (r // dh == c).astype(F32)

    def headln(z, w_t, b_t):
        s1 = lax.dot_general(z, seg, (((1,), (0,)), ((), ())),
                             preferred_element_type=F32)
        m = s1 * (1.0 / dh)
        s2 = lax.dot_general(z * z, seg, (((1,), (0,)), ((), ())),
                             preferred_element_type=F32)
        vv = s2 * (1.0 / dh) - m * m
        rstd = lax.rsqrt(vv + EPS)
        m_e = lax.dot_general(m, seg, (((1,), (1,)), ((), ())),
                              preferred_element_type=F32)
        r_e = lax.dot_general(rstd, seg, (((1,), (1,)), ((), ())),
                              preferred_element_type=F32)
        return (z - m_e) * r_e * w_t + b_t

    scale = 1.0 / (dh ** 0.5)
    qn = headln(q, qnw_ref[...], qnb_ref[...]) * scale
    kn = headln(k, knw_ref[...], knb_ref[...])
    o_ref[...] = jnp.concatenate([qn, kn, v], axis=1)


def _k1(xf, shift, lnw, lnb, wqkv, qnw_t, qnb_t, knw_t, knb_t, *, rows_pb, d, h, dh):
    n = xf.shape[0]
    grid = (n // rows_pb,)
    spb = n // shift.shape[0] // rows_pb
    return pl.pallas_call(
        functools.partial(_k1_body, d=d, h=h, dh=dh, spb=spb),
        grid=grid,
        in_specs=[
            pl.BlockSpec((rows_pb, d), lambda i: (i, 0)),
            pl.BlockSpec(shift.shape, lambda i: (0, 0)),
            pl.BlockSpec((d,), lambda i: (0,)),
            pl.BlockSpec((d,), lambda i: (0,)),
            pl.BlockSpec((3 * d, d), lambda i: (0, 0)),
            pl.BlockSpec((d,), lambda i: (0,)),
            pl.BlockSpec((d,), lambda i: (0,)),
            pl.BlockSpec((d,), lambda i: (0,)),
            pl.BlockSpec((d,), lambda i: (0,)),
        ],
        out_specs=pl.BlockSpec((rows_pb, 3 * d), lambda i: (i, 0)),
        out_shape=jax.ShapeDtypeStruct((n, 3 * d), F32),
        interpret=_INTERPRET,
    )(xf, shift, lnw, lnb, wqkv, qnw_t, qnb_t, knw_t, knb_t)


def _k2_body(q_ref, k_ref, v_ref, o_ref, *, h, dh):
    q = q_ref[...]
    k = k_ref[...]
    v = v_ref[...]
    outs = []
    for hi in range(h):
        qh = q[:, hi * dh:(hi + 1) * dh]
        kh = k[:, hi * dh:(hi + 1) * dh]
        vh = v[:, hi * dh:(hi + 1) * dh]
        s = lax.dot_general(qh, kh, (((1,), (1,)), ((), ())),
                            preferred_element_type=F32)
        m = jnp.max(s, axis=1, keepdims=True)
        p = jnp.exp(s - m)
        p = p / jnp.sum(p, axis=1, keepdims=True)
        outs.append(lax.dot_general(p, vh, (((1,), (0,)), ((), ())),
                                    preferred_element_type=F32))
    o_ref[...] = jnp.concatenate(outs, axis=1)


def _k2(qkvb, *, b, s, d, h, dh, q_pb):
    n = b * s
    nqb = s // q_pb
    return pl.pallas_call(
        functools.partial(_k2_body, h=h, dh=dh),
        grid=(b, nqb),
        in_specs=[
            pl.BlockSpec((q_pb, d), lambda bi, qi: (bi * nqb + qi, 0)),
            pl.BlockSpec((s, d), lambda bi, qi: (bi, 1)),
            pl.BlockSpec((s, d), lambda bi, qi: (bi, 2)),
        ],
        out_specs=pl.BlockSpec((q_pb, d), lambda bi, qi: (bi * nqb + qi, 0)),
        out_shape=jax.ShapeDtypeStruct((n, d), F32),
        interpret=_INTERPRET,
    )(qkvb, qkvb, qkvb)


def _k3_body(x_ref, o_ref, wout_ref, wg_ref, xf_ref, pt_ref):
    o = o_ref[...]
    xf = x_ref[...] + lax.dot_general(o, wout_ref[...], (((1,), (1,)), ((), ())),
                                      preferred_element_type=F32)
    xf_ref[...] = xf
    st = lax.dot_general(wg_ref[...], xf, (((1,), (1,)), ((), ())),
                         preferred_element_type=F32)
    m = jnp.max(st, axis=0, keepdims=True)
    p = jnp.exp(st - m)
    pt_ref[...] = p / jnp.sum(p, axis=0, keepdims=True)


def _k3(xf0, o, wout, wg, *, rows_pb, d, e):
    n = xf0.shape[0]
    grid = (n // rows_pb,)
    return pl.pallas_call(
        _k3_body,
        grid=grid,
        in_specs=[
            pl.BlockSpec((rows_pb, d), lambda i: (i, 0)),
            pl.BlockSpec((rows_pb, d), lambda i: (i, 0)),
            pl.BlockSpec((d, d), lambda i: (0, 0)),
            pl.BlockSpec((e, d), lambda i: (0, 0)),
        ],
        out_specs=[
            pl.BlockSpec((rows_pb, d), lambda i: (i, 0)),
            pl.BlockSpec((e, rows_pb), lambda i: (0, i)),
        ],
        out_shape=[
            jax.ShapeDtypeStruct((n, d), F32),
            jax.ShapeDtypeStruct((e, n), F32),
        ],
        interpret=_INTERPRET,
    )(xf0, o, wout, wg)


def _k4_body(pt_ref, thr_ref, ngt_ref, *, kk, e):
    bits = lax.bitcast_convert_type(pt_ref[...], jnp.int32)

    def step(it, cur):
        cand = cur | (1 << (30 - it))
        cnt = jnp.sum((bits >= cand).astype(jnp.int32), axis=1, keepdims=True)
        return jnp.where(cnt >= kk, cand, cur)

    thr = lax.fori_loop(0, 31, step, jnp.zeros((e, 1), jnp.int32))
    ngt = jnp.sum((bits > thr).astype(jnp.int32), axis=1, keepdims=True)
    pad = jnp.zeros((16 - e, 1), jnp.int32)
    thr_ref[...] = jnp.concatenate([thr, pad], axis=0).reshape(16)
    ngt_ref[...] = jnp.concatenate([ngt, pad], axis=0).reshape(16)


def _k4(probs_t, *, kk):
    e, n = probs_t.shape
    return pl.pallas_call(
        functools.partial(_k4_body, kk=kk, e=e),
        out_shape=[
            jax.ShapeDtypeStruct((16,), jnp.int32),
            jax.ShapeDtypeStruct((16,), jnp.int32),
        ],
        interpret=_INTERPRET,
    )(probs_t)


def _k5_body(y0_ref, lnw_ref, lnb_ref, fc1_ref, fc2_ref, b1_ref, b2_ref,
             w_ref, o_ref, xn_scr, *, ndd):
    dd = pl.program_id(1)

    @pl.when(dd == 0)
    def _():
        z = y0_ref[...]
        mu = jnp.mean(z, axis=1, keepdims=True)
        zc = z - mu
        var = jnp.mean(zc * zc, axis=1, keepdims=True)
        xn_scr[...] = (zc * lax.rsqrt(var + EPS) * lnw_ref[...]
                       + lnb_ref[...]).astype(BF16)

    h1 = lax.dot_general(xn_scr[...], fc1_ref[0], (((1,), (1,)), ((), ())),
                         preferred_element_type=F32) + b1_ref[0, 0]
    g = 0.5 * h1 * (1.0 + jnp.tanh(0.7978845608028654
                                   * (h1 + 0.044715 * h1 * h1 * h1)))
    part = lax.dot_general(g.astype(BF16), fc2_ref[0], (((1,), (1,)), ((), ())),
                           preferred_element_type=F32)

    @pl.when(dd == 0)
    def _():
        o_ref[...] = part

    @pl.when(dd != 0)
    def _():
        o_ref[...] += part

    @pl.when(dd == ndd - 1)
    def _():
        o_ref[...] = (o_ref[...] + b2_ref[0, 0]) * w_ref[0, 0][:, None]


def _k5(y0, lnw, lnb, fc1b, fc2b, b1s, b2s, wsel, *, kk, d, e, dd_pb):
    ddim = fc1b.shape[1]
    ndd = ddim // dd_pb
    return pl.pallas_call(
        functools.partial(_k5_body, ndd=ndd),
        grid=(e, ndd),
        in_specs=[
            pl.BlockSpec((kk, d), lambda ei, di: (ei, 0)),
            pl.BlockSpec((d,), lambda ei, di: (0,)),
            pl.BlockSpec((d,), lambda ei, di: (0,)),
            pl.BlockSpec((1, dd_pb, d), lambda ei, di: (ei, di, 0)),
            pl.BlockSpec((1, d, dd_pb), lambda ei, di: (ei, 0, di)),
            pl.BlockSpec((1, 1, dd_pb), lambda ei, di: (ei * ndd + di, 0, 0)),
            pl.BlockSpec((1, 1, d), lambda ei, di: (ei, 0, 0)),
            pl.BlockSpec((1, 1, kk), lambda ei, di: (ei, 0, 0)),
        ],
        out_specs=pl.BlockSpec((kk, d), lambda ei, di: (ei, 0)),
        out_shape=jax.ShapeDtypeStruct((kk * e, d), F32),
        scratch_shapes=[pltpu.VMEM((kk, d), BF16)],
        compiler_params=pltpu.CompilerParams(
            dimension_semantics=("arbitrary", "arbitrary")),
        interpret=_INTERPRET,
    )(y0, lnw, lnb, fc1b, fc2b,
      b1s.reshape(e * (ddim // dd_pb), 1, dd_pb), b2s.reshape(e, 1, d),
      wsel.reshape(e, 1, kk))


def _route_stub(probs_t, thr, ngt, *, kk):
    e, n = probs_t.shape
    bits = lax.bitcast_convert_type(probs_t, jnp.int32)
    thr_ = thr[:e, None]
    gt = bits > thr_
    eq = bits == thr_
    need = (kk - ngt[:e])[:, None]
    tie_rank = jnp.cumsum(eq.astype(jnp.int32), axis=1)
    take = gt | (eq & (tie_rank <= need))
    order = jnp.argsort(~take, axis=1, stable=True)
    idx = order[:, :kk].astype(jnp.int32)
    wsel = jnp.take_along_axis(probs_t, idx, axis=1)
    return idx, wsel


def _gather_stub(xf, idx_flat):
    return jnp.take(xf, idx_flat, axis=0)


def _scatter_stub(xf, y, idx_flat):
    return xf.at[idx_flat].add(y)


def kernel(x, t, ada_w, ada_b, ada_ln_w, ada_ln_b, Wqkv, qn_w, qn_b, kn_w, kn_b,
           Wout, mlp_ln_w, mlp_ln_b, Wg, fc1s, fc2s, b1s, b2s):
    b, s, d = x.shape
    e, ddim, _ = fc1s.shape
    dh = qn_w.shape[0]
    h = Wqkv.shape[0] // (3 * dh)
    n = b * s
    kk = n // e

    xf0 = x.reshape(n, d)
    shift = _k0(t.reshape(b, d), ada_w, ada_b)
    qkvb = _k1(xf0, shift, ada_ln_w, ada_ln_b, Wqkv,
               jnp.tile(qn_w, h), jnp.tile(qn_b, h),
               jnp.tile(kn_w, h), jnp.tile(kn_b, h),
               rows_pb=512, d=d, h=h, dh=dh)
    o = _k2(qkvb, b=b, s=s, d=d, h=h, dh=dh, q_pb=512)
    xf, probs_t = _k3(xf0, o, Wout, Wg, rows_pb=512, d=d, e=e)
    thr, ngt = _k4(probs_t, kk=kk)

    idx, wsel = _route_stub(probs_t, thr, ngt, kk=kk)
    idx_flat = idx.reshape(-1)
    y0 = _gather_stub(xf, idx_flat)

    y = _k5(y0, mlp_ln_w, mlp_ln_b,
            fc1s.astype(BF16), fc2s.astype(BF16), b1s, b2s, wsel,
            kk=kk, d=d, e=e, dd_pb=1024)

    out = _scatter_stub(xf, y, idx_flat)
    return out.reshape(b, s, d)


# speedup vs baseline: 1.3420x; 1.0186x over previous
import functools

import jax
import jax.numpy as jnp
from jax import lax
from jax.experimental import pallas as pl
from jax.experimental.pallas import tpu as pltpu
from jax.experimental.pallas import tpu_sc as plsc

F32 = jnp.float32
BF16 = jnp.bfloat16
EPS = 1e-5
_INTERPRET = False


def _k0_body(t_ref, w_ref, b_ref, o_ref):
    t = t_ref[...]
    s = t * jax.nn.sigmoid(t)
    o_ref[...] = (
        lax.dot_general(s, w_ref[...], (((1,), (1,)), ((), ())),
                        preferred_element_type=F32)
        + b_ref[...]
    )


def _k0(t2, ada_w, ada_b):
    return pl.pallas_call(
        _k0_body,
        out_shape=jax.ShapeDtypeStruct(t2.shape, F32),
        interpret=_INTERPRET,
    )(t2, ada_w, ada_b)


def _k1_body(x_ref, sh_ref, lnw_ref, lnb_ref, wqkv_ref,
             qnw_ref, qnb_ref, knw_ref, knb_ref, o_ref, *, d, h, dh, spb):
    x = x_ref[...]
    mu = jnp.mean(x, axis=1, keepdims=True)
    xc = x - mu
    var = jnp.mean(xc * xc, axis=1, keepdims=True)
    sh = sh_ref[pl.ds(pl.program_id(0) // spb, 1), :]
    hpre = xc * lax.rsqrt(var + EPS) * lnw_ref[...] + lnb_ref[...] + sh
    qkv = lax.dot_general(hpre, wqkv_ref[...], (((1,), (1,)), ((), ())),
                          preferred_element_type=F32)
    q = qkv[:, :d]
    k = qkv[:, d:2 * d]
    v = qkv[:, 2 * d:]
    r = lax.broadcasted_iota(jnp.int32, (d, h), 0)
    c = lax.broadcasted_iota(jnp.int32, (d, h), 1)
    seg = (r // dh == c).astype(F32)

    def headln(z, w_t, b_t):
        s1 = lax.dot_general(z, seg, (((1,), (0,)), ((), ())),
                             preferred_element_type=F32)
        m = s1 * (1.0 / dh)
        s2 = lax.dot_general(z * z, seg, (((1,), (0,)), ((), ())),
                             preferred_element_type=F32)
        vv = s2 * (1.0 / dh) - m * m
        rstd = lax.rsqrt(vv + EPS)
        m_e = lax.dot_general(m, seg, (((1,), (1,)), ((), ())),
                              preferred_element_type=F32)
        r_e = lax.dot_general(rstd, seg, (((1,), (1,)), ((), ())),
                              preferred_element_type=F32)
        return (z - m_e) * r_e * w_t + b_t

    scale = 1.0 / (dh ** 0.5)
    qn = headln(q, qnw_ref[...], qnb_ref[...]) * scale
    kn = headln(k, knw_ref[...], knb_ref[...])
    o_ref[...] = jnp.concatenate([qn, kn, v], axis=1)


def _k1(xf, shift, lnw, lnb, wqkv, qnw_t, qnb_t, knw_t, knb_t, *, rows_pb, d, h, dh):
    n = xf.shape[0]
    grid = (n // rows_pb,)
    spb = n // shift.shape[0] // rows_pb
    return pl.pallas_call(
        functools.partial(_k1_body, d=d, h=h, dh=dh, spb=spb),
        grid=grid,
        in_specs=[
            pl.BlockSpec((rows_pb, d), lambda i: (i, 0)),
            pl.BlockSpec(shift.shape, lambda i: (0, 0)),
            pl.BlockSpec((d,), lambda i: (0,)),
            pl.BlockSpec((d,), lambda i: (0,)),
            pl.BlockSpec((3 * d, d), lambda i: (0, 0)),
            pl.BlockSpec((d,), lambda i: (0,)),
            pl.BlockSpec((d,), lambda i: (0,)),
            pl.BlockSpec((d,), lambda i: (0,)),
            pl.BlockSpec((d,), lambda i: (0,)),
        ],
        out_specs=pl.BlockSpec((rows_pb, 3 * d), lambda i: (i, 0)),
        out_shape=jax.ShapeDtypeStruct((n, 3 * d), F32),
        interpret=_INTERPRET,
    )(xf, shift, lnw, lnb, wqkv, qnw_t, qnb_t, knw_t, knb_t)


def _k2_body(q_ref, k_ref, v_ref, o_ref, *, h, dh):
    q = q_ref[...]
    k = k_ref[...]
    v = v_ref[...]
    outs = []
    for hi in range(h):
        qh = q[:, hi * dh:(hi + 1) * dh]
        kh = k[:, hi * dh:(hi + 1) * dh]
        vh = v[:, hi * dh:(hi + 1) * dh]
        s = lax.dot_general(qh, kh, (((1,), (1,)), ((), ())),
                            preferred_element_type=F32)
        m = jnp.max(s, axis=1, keepdims=True)
        p = jnp.exp(s - m)
        p = p / jnp.sum(p, axis=1, keepdims=True)
        outs.append(lax.dot_general(p, vh, (((1,), (0,)), ((), ())),
                                    preferred_element_type=F32))
    o_ref[...] = jnp.concatenate(outs, axis=1)


def _k2(qkvb, *, b, s, d, h, dh, q_pb):
    n = b * s
    nqb = s // q_pb
    return pl.pallas_call(
        functools.partial(_k2_body, h=h, dh=dh),
        grid=(b, nqb),
        in_specs=[
            pl.BlockSpec((q_pb, d), lambda bi, qi: (bi * nqb + qi, 0)),
            pl.BlockSpec((s, d), lambda bi, qi: (bi, 1)),
            pl.BlockSpec((s, d), lambda bi, qi: (bi, 2)),
        ],
        out_specs=pl.BlockSpec((q_pb, d), lambda bi, qi: (bi * nqb + qi, 0)),
        out_shape=jax.ShapeDtypeStruct((n, d), F32),
        interpret=_INTERPRET,
    )(qkvb, qkvb, qkvb)


def _k3_body(x_ref, o_ref, wout_ref, wg_ref, xf_ref, pt_ref):
    o = o_ref[...]
    xf = x_ref[...] + lax.dot_general(o, wout_ref[...], (((1,), (1,)), ((), ())),
                                      preferred_element_type=F32)
    xf_ref[...] = xf
    st = lax.dot_general(wg_ref[...], xf, (((1,), (1,)), ((), ())),
                         preferred_element_type=F32)
    m = jnp.max(st, axis=0, keepdims=True)
    p = jnp.exp(st - m)
    pt_ref[...] = p / jnp.sum(p, axis=0, keepdims=True)


def _k3(xf0, o, wout, wg, *, rows_pb, d, e):
    n = xf0.shape[0]
    grid = (n // rows_pb,)
    return pl.pallas_call(
        _k3_body,
        grid=grid,
        in_specs=[
            pl.BlockSpec((rows_pb, d), lambda i: (i, 0)),
            pl.BlockSpec((rows_pb, d), lambda i: (i, 0)),
            pl.BlockSpec((d, d), lambda i: (0, 0)),
            pl.BlockSpec((e, d), lambda i: (0, 0)),
        ],
        out_specs=[
            pl.BlockSpec((rows_pb, d), lambda i: (i, 0)),
            pl.BlockSpec((e, rows_pb), lambda i: (0, i)),
        ],
        out_shape=[
            jax.ShapeDtypeStruct((n, d), F32),
            jax.ShapeDtypeStruct((e, n), F32),
        ],
        interpret=_INTERPRET,
    )(xf0, o, wout, wg)


def _k4_body(pt_ref, thr_ref, ngt_ref, *, kk, e):
    bits = lax.bitcast_convert_type(pt_ref[...], jnp.int32)

    def step(it, cur):
        cand = cur | (1 << (30 - it))
        cnt = jnp.sum((bits >= cand).astype(jnp.int32), axis=1, keepdims=True)
        return jnp.where(cnt >= kk, cand, cur)

    thr = lax.fori_loop(0, 31, step, jnp.zeros((e, 1), jnp.int32))
    ngt = jnp.sum((bits > thr).astype(jnp.int32), axis=1, keepdims=True)
    pad = jnp.zeros((16 - e, 1), jnp.int32)
    thr_ref[...] = lax.bitcast_convert_type(
        jnp.concatenate([thr, pad], axis=0).reshape(16), F32)
    ngt_ref[...] = jnp.concatenate([ngt, pad], axis=0).reshape(16)


def _k4(probs_t, *, kk):
    e, n = probs_t.shape
    return pl.pallas_call(
        functools.partial(_k4_body, kk=kk, e=e),
        out_shape=[
            jax.ShapeDtypeStruct((16,), F32),
            jax.ShapeDtypeStruct((16,), jnp.int32),
        ],
        interpret=_INTERPRET,
    )(probs_t)


def _k5_body(y0_ref, lnw_ref, lnb_ref, fc1_ref, fc2_ref, b1_ref, b2_ref,
             w_ref, o_ref, xn_scr, *, ndd):
    dd = pl.program_id(1)

    @pl.when(dd == 0)
    def _():
        z = y0_ref[...]
        mu = jnp.mean(z, axis=1, keepdims=True)
        zc = z - mu
        var = jnp.mean(zc * zc, axis=1, keepdims=True)
        xn_scr[...] = (zc * lax.rsqrt(var + EPS) * lnw_ref[...]
                       + lnb_ref[...]).astype(BF16)

    h1 = lax.dot_general(xn_scr[...], fc1_ref[0], (((1,), (1,)), ((), ())),
                         preferred_element_type=F32) + b1_ref[0, 0]
    g = 0.5 * h1 * (1.0 + jnp.tanh(0.7978845608028654
                                   * (h1 + 0.044715 * h1 * h1 * h1)))
    part = lax.dot_general(g.astype(BF16), fc2_ref[0], (((1,), (1,)), ((), ())),
                           preferred_element_type=F32)

    @pl.when(dd == 0)
    def _():
        o_ref[...] = part

    @pl.when(dd != 0)
    def _():
        o_ref[...] += part

    @pl.when(dd == ndd - 1)
    def _():
        o_ref[...] = (o_ref[...] + b2_ref[0, 0]) * w_ref[0, 0][:, None]


def _k5(y0, lnw, lnb, fc1b, fc2b, b1s, b2s, wsel, *, kk, d, e, dd_pb):
    ddim = fc1b.shape[1]
    ndd = ddim // dd_pb
    return pl.pallas_call(
        functools.partial(_k5_body, ndd=ndd),
        grid=(e, ndd),
        in_specs=[
            pl.BlockSpec((kk, d), lambda ei, di: (ei, 0)),
            pl.BlockSpec((d,), lambda ei, di: (0,)),
            pl.BlockSpec((d,), lambda ei, di: (0,)),
            pl.BlockSpec((1, dd_pb, d), lambda ei, di: (ei, di, 0)),
            pl.BlockSpec((1, d, dd_pb), lambda ei, di: (ei, 0, di)),
            pl.BlockSpec((1, 1, dd_pb), lambda ei, di: (ei * ndd + di, 0, 0)),
            pl.BlockSpec((1, 1, d), lambda ei, di: (ei, 0, 0)),
            pl.BlockSpec((1, 1, kk), lambda ei, di: (ei, 0, 0)),
        ],
        out_specs=pl.BlockSpec((kk, d), lambda ei, di: (ei, 0)),
        out_shape=jax.ShapeDtypeStruct((kk * e, d), F32),
        scratch_shapes=[pltpu.VMEM((kk, d), BF16)],
        compiler_params=pltpu.CompilerParams(
            dimension_semantics=("arbitrary", "arbitrary")),
        interpret=_INTERPRET,
    )(y0, lnw, lnb, fc1b, fc2b,
      b1s.reshape(e * (ddim // dd_pb), 1, dd_pb), b2s.reshape(e, 1, d),
      wsel.reshape(e, 1, kk))


def _sca(probs_t, thr, ngt, *, kk):
    e, n = probs_t.shape
    mesh = plsc.VectorSubcoreMesh(core_axis_name="c", subcore_axis_name="s")

    @functools.partial(
        pl.kernel, mesh=mesh,
        compiler_params=pltpu.CompilerParams(needs_layout_passes=False),
        out_type=[jax.ShapeDtypeStruct((e, kk), jnp.int32),
                  jax.ShapeDtypeStruct((e, kk), F32)],
        scratch_types=[pltpu.VMEM((n,), F32),
                       pltpu.VMEM((16,), F32),
                       pltpu.VMEM((16,), jnp.int32),
                       pltpu.VMEM((kk + 16,), jnp.int32),
                       pltpu.VMEM((kk + 16,), F32)],
    )
    def k(pt_hbm, thr_hbm, ngt_hbm, idx_hbm, w_hbm, p_v, t_v, g_v, ib, wb):
        wid = lax.axis_index("s") * 2 + lax.axis_index("c")
        lanes = lax.iota(jnp.int32, 16)

        def dyng(vec, idx):
            return lax.gather(
                vec, idx[:, None],
                lax.GatherDimensionNumbers(offset_dims=(),
                                           collapsed_slice_dims=(0,),
                                           start_index_map=(0,)),
                (1,), mode=lax.GatherScatterMode.PROMISE_IN_BOUNDS)

        def bcast(vec, lane):
            return dyng(vec, jnp.zeros((16,), jnp.int32) + lane)

        def prefix16(x):
            for sh in (1, 2, 4, 8):
                shifted = dyng(x, jnp.maximum(lanes - sh, 0))
                x = x + jnp.where(lanes >= sh, shifted, 0)
            return x

        @pl.when(wid < e)
        def _():
            pltpu.sync_copy(pt_hbm.at[wid], p_v)
            pltpu.sync_copy(thr_hbm, t_v)
            pltpu.sync_copy(ngt_hbm, g_v)
            thr_v16 = bcast(t_v[...], wid)
            need_v16 = kk - bcast(g_v[...], wid)

            def body(c, carry):
                nt, ne = carry
                p = p_v[pl.ds(c * 16, 16)]
                gt = p > thr_v16
                eq = p == thr_v16
                eqp = prefix16(jnp.where(eq, 1, 0))
                take = gt | (eq & ((ne + eqp) <= need_v16))
                tkp = prefix16(jnp.where(take, 1, 0))
                pos = nt + tkp - 1
                plsc.store_scatter(ib, [pos], c * 16 + lanes, mask=take)
                plsc.store_scatter(wb, [pos], p, mask=take)
                nt = nt + bcast(tkp, 15)
                ne = ne + bcast(eqp, 15)
                return nt, ne

            z = jnp.zeros((16,), jnp.int32)
            lax.fori_loop(0, n // 16, body, (z, z))
            pltpu.sync_copy(ib.at[pl.ds(0, kk)], idx_hbm.at[wid])
            pltpu.sync_copy(wb.at[pl.ds(0, kk)], w_hbm.at[wid])

    return k(probs_t, thr, ngt)


def _scb(xf, idx_flat):
    n_tok, d = xf.shape
    rpw = n_tok // 32
    half = rpw // 2
    mesh = plsc.VectorSubcoreMesh(core_axis_name="c", subcore_axis_name="s")

    @functools.partial(
        pl.kernel, mesh=mesh,
        compiler_params=pltpu.CompilerParams(needs_layout_passes=False),
        out_type=jax.ShapeDtypeStruct((n_tok, d), F32),
        scratch_types=[pltpu.VMEM((2, half), jnp.int32),
                       pltpu.VMEM((half, d), F32),
                       pltpu.SemaphoreType.DMA],
    )
    def k(xf_hbm, idx_hbm, y0_hbm, ib, rows, sem):
        wid = lax.axis_index("s") * 2 + lax.axis_index("c")
        base = wid * rpw
        pltpu.sync_copy(idx_hbm.at[pl.ds(base, half)], ib.at[0])
        pltpu.sync_copy(idx_hbm.at[pl.ds(base + half, half)], ib.at[1])
        for j in range(2):
            pltpu.async_copy(xf_hbm.at[ib.at[j]], rows, sem).wait()
            pltpu.sync_copy(rows, y0_hbm.at[pl.ds(base + j * half, half)])

    return k(xf, idx_flat)


def _k6_body(xf_ref, idxf_ref, y_ref, o_ref, *, rows_pb):
    tok0 = pl.program_id(0) * rows_pb
    nslots = idxf_ref.shape[1]
    rows = tok0 + lax.broadcasted_iota(jnp.int32, (rows_pb, nslots), 0)
    p = jnp.where(idxf_ref[...] == rows, 1.0, 0.0).astype(BF16)
    delta = lax.dot_general(p, y_ref[...], (((1,), (0,)), ((), ())),
                            preferred_element_type=F32)
    o_ref[...] = xf_ref[...] + delta


def _k6(xf, idx_flat, y, *, rows_pb):
    n_tok, d = xf.shape
    ns = idx_flat.shape[0]
    return pl.pallas_call(
        functools.partial(_k6_body, rows_pb=rows_pb),
        grid=(n_tok // rows_pb,),
        in_specs=[
            pl.BlockSpec((rows_pb, d), lambda i: (i, 0)),
            pl.BlockSpec((1, ns), lambda i: (0, 0)),
            pl.BlockSpec((ns, d), lambda i: (0, 0)),
        ],
        out_specs=pl.BlockSpec((rows_pb, d), lambda i: (i, 0)),
        out_shape=jax.ShapeDtypeStruct((n_tok, d), F32),
        interpret=_INTERPRET,
    )(xf, idx_flat.reshape(1, ns), y)


def kernel(x, t, ada_w, ada_b, ada_ln_w, ada_ln_b, Wqkv, qn_w, qn_b, kn_w, kn_b,
           Wout, mlp_ln_w, mlp_ln_b, Wg, fc1s, fc2s, b1s, b2s):
    b, s, d = x.shape
    e, ddim, _ = fc1s.shape
    dh = qn_w.shape[0]
    h = Wqkv.shape[0] // (3 * dh)
    n = b * s
    kk = n // e

    xf0 = x.reshape(n, d)
    shift = _k0(t.reshape(b, d), ada_w, ada_b)
    qkvb = _k1(xf0, shift, ada_ln_w, ada_ln_b, Wqkv,
               jnp.tile(qn_w, h), jnp.tile(qn_b, h),
               jnp.tile(kn_w, h), jnp.tile(kn_b, h),
               rows_pb=512, d=d, h=h, dh=dh)
    o = _k2(qkvb, b=b, s=s, d=d, h=h, dh=dh, q_pb=512)
    xf, probs_t = _k3(xf0, o, Wout, Wg, rows_pb=512, d=d, e=e)
    thr, ngt = _k4(probs_t, kk=kk)

    idx, wsel = _sca(probs_t, thr, ngt, kk=kk)
    idx_flat = idx.reshape(-1)
    y0 = _scb(xf, idx_flat)

    y = _k5(y0, mlp_ln_w, mlp_ln_b,
            fc1s.astype(BF16), fc2s.astype(BF16), b1s, b2s, wsel,
            kk=kk, d=d, e=e, dd_pb=1024)

    out = _k6(xf, idx_flat, y.astype(BF16), rows_pb=512)
    return out.reshape(b, s, d)


# speedup vs baseline: 1.8951x; 1.4121x over previous
import functools

import jax
import jax.numpy as jnp
from jax import lax
from jax.experimental import pallas as pl
from jax.experimental.pallas import tpu as pltpu
from jax.experimental.pallas import tpu_sc as plsc

F32 = jnp.float32
BF16 = jnp.bfloat16
EPS = 1e-5
_INTERPRET = False


def _k0_body(t_ref, w_ref, b_ref, o_ref):
    t = t_ref[...]
    s = t * jax.nn.sigmoid(t)
    o_ref[...] = (
        lax.dot_general(s, w_ref[...], (((1,), (1,)), ((), ())),
                        preferred_element_type=F32)
        + b_ref[...]
    )


def _k0(t2, ada_w, ada_b):
    return pl.pallas_call(
        _k0_body,
        out_shape=jax.ShapeDtypeStruct(t2.shape, F32),
        interpret=_INTERPRET,
    )(t2, ada_w, ada_b)


def _k1_body(x_ref, sh_ref, lnw_ref, lnb_ref, wqkv_ref,
             qnw_ref, qnb_ref, knw_ref, knb_ref, o_ref, *, d, h, dh, spb):
    x = x_ref[...]
    mu = jnp.mean(x, axis=1, keepdims=True)
    xc = x - mu
    var = jnp.mean(xc * xc, axis=1, keepdims=True)
    sh = sh_ref[pl.ds(pl.program_id(0) // spb, 1), :]
    hpre = xc * lax.rsqrt(var + EPS) * lnw_ref[...] + lnb_ref[...] + sh
    qkv = lax.dot_general(hpre, wqkv_ref[...], (((1,), (1,)), ((), ())),
                          preferred_element_type=F32)
    q = qkv[:, :d]
    k = qkv[:, d:2 * d]
    v = qkv[:, 2 * d:]
    r = lax.broadcasted_iota(jnp.int32, (d, h), 0)
    c = lax.broadcasted_iota(jnp.int32, (d, h), 1)
    seg = (r // dh == c).astype(F32)

    def headln(z, w_t, b_t):
        s1 = lax.dot_general(z, seg, (((1,), (0,)), ((), ())),
                             preferred_element_type=F32)
        m = s1 * (1.0 / dh)
        s2 = lax.dot_general(z * z, seg, (((1,), (0,)), ((), ())),
                             preferred_element_type=F32)
        vv = s2 * (1.0 / dh) - m * m
        rstd = lax.rsqrt(vv + EPS)
        m_e = lax.dot_general(m, seg, (((1,), (1,)), ((), ())),
                              preferred_element_type=F32)
        r_e = lax.dot_general(rstd, seg, (((1,), (1,)), ((), ())),
                              preferred_element_type=F32)
        return (z - m_e) * r_e * w_t + b_t

    scale = 1.0 / (dh ** 0.5)
    qn = headln(q, qnw_ref[...], qnb_ref[...]) * scale
    kn = headln(k, knw_ref[...], knb_ref[...])
    o_ref[...] = jnp.concatenate([qn, kn, v], axis=1)


def _k1(xf, shift, lnw, lnb, wqkv, qnw_t, qnb_t, knw_t, knb_t, *, rows_pb, d, h, dh):
    n = xf.shape[0]
    grid = (n // rows_pb,)
    spb = n // shift.shape[0] // rows_pb
    return pl.pallas_call(
        functools.partial(_k1_body, d=d, h=h, dh=dh, spb=spb),
        grid=grid,
        in_specs=[
            pl.BlockSpec((rows_pb, d), lambda i: (i, 0)),
            pl.BlockSpec(shift.shape, lambda i: (0, 0)),
            pl.BlockSpec((d,), lambda i: (0,)),
            pl.BlockSpec((d,), lambda i: (0,)),
            pl.BlockSpec((3 * d, d), lambda i: (0, 0)),
            pl.BlockSpec((d,), lambda i: (0,)),
            pl.BlockSpec((d,), lambda i: (0,)),
            pl.BlockSpec((d,), lambda i: (0,)),
            pl.BlockSpec((d,), lambda i: (0,)),
        ],
        out_specs=pl.BlockSpec((rows_pb, 3 * d), lambda i: (i, 0)),
        out_shape=jax.ShapeDtypeStruct((n, 3 * d), F32),
        interpret=_INTERPRET,
    )(xf, shift, lnw, lnb, wqkv, qnw_t, qnb_t, knw_t, knb_t)


def _k2_body(x_ref, q_ref, k_ref, v_ref, wout_ref, wg_ref, xf_ref, pt_ref,
             *, h, dh):
    q = q_ref[...]
    k = k_ref[...]
    v = v_ref[...]
    outs = []
    for hi in range(h):
        qh = q[:, hi * dh:(hi + 1) * dh]
        kh = k[:, hi * dh:(hi + 1) * dh]
        vh = v[:, hi * dh:(hi + 1) * dh]
        s = lax.dot_general(qh, kh, (((1,), (1,)), ((), ())),
                            preferred_element_type=F32)
        p = jnp.exp(s)
        l = jnp.sum(p, axis=1, keepdims=True)
        o = lax.dot_general(p, vh, (((1,), (0,)), ((), ())),
                            preferred_element_type=F32)
        outs.append(o / l)
    o = jnp.concatenate(outs, axis=1)
    xf = x_ref[...] + lax.dot_general(o, wout_ref[...],
                                      (((1,), (1,)), ((), ())),
                                      preferred_element_type=F32)
    xf_ref[...] = xf
    st = lax.dot_general(wg_ref[...], xf, (((1,), (1,)), ((), ())),
                         preferred_element_type=F32)
    m = jnp.max(st, axis=0, keepdims=True)
    p = jnp.exp(st - m)
    pt_ref[...] = p / jnp.sum(p, axis=0, keepdims=True)


def _k2(xf0, qkvb, wout, wg, *, b, s, d, h, dh, q_pb, e):
    n = b * s
    nqb = s // q_pb
    return pl.pallas_call(
        functools.partial(_k2_body, h=h, dh=dh),
        grid=(b, nqb),
        in_specs=[
            pl.BlockSpec((q_pb, d), lambda bi, qi: (bi * nqb + qi, 0)),
            pl.BlockSpec((q_pb, d), lambda bi, qi: (bi * nqb + qi, 0)),
            pl.BlockSpec((s, d), lambda bi, qi: (bi, 1)),
            pl.BlockSpec((s, d), lambda bi, qi: (bi, 2)),
            pl.BlockSpec((d, d), lambda bi, qi: (0, 0)),
            pl.BlockSpec((e, d), lambda bi, qi: (0, 0)),
        ],
        out_specs=[
            pl.BlockSpec((q_pb, d), lambda bi, qi: (bi * nqb + qi, 0)),
            pl.BlockSpec((e, q_pb), lambda bi, qi: (0, bi * nqb + qi)),
        ],
        out_shape=[
            jax.ShapeDtypeStruct((n, d), F32),
            jax.ShapeDtypeStruct((e, n), F32),
        ],
        compiler_params=pltpu.CompilerParams(vmem_limit_bytes=66_500_000),
        interpret=_INTERPRET,
    )(xf0, qkvb, qkvb, qkvb, wout, wg)


def _k4_body(pt_ref, thr_ref, ngt_ref, *, kk, e):
    bits = lax.bitcast_convert_type(pt_ref[...], jnp.int32)

    def step(it, cur):
        cand = cur | (1 << (30 - it))
        cnt = jnp.sum((bits >= cand).astype(jnp.int32), axis=1, keepdims=True)
        return jnp.where(cnt >= kk, cand, cur)

    thr = lax.fori_loop(0, 31, step, jnp.zeros((e, 1), jnp.int32))
    ngt = jnp.sum((bits > thr).astype(jnp.int32), axis=1, keepdims=True)
    pad = jnp.zeros((16 - e, 1), jnp.int32)
    thr_ref[...] = lax.bitcast_convert_type(
        jnp.concatenate([thr, pad], axis=0).reshape(16), F32)
    ngt_ref[...] = jnp.concatenate([ngt, pad], axis=0).reshape(16)


def _k4(probs_t, *, kk):
    e, n = probs_t.shape
    return pl.pallas_call(
        functools.partial(_k4_body, kk=kk, e=e),
        out_shape=[
            jax.ShapeDtypeStruct((16,), F32),
            jax.ShapeDtypeStruct((16,), jnp.int32),
        ],
        interpret=_INTERPRET,
    )(probs_t)


def _k5_body(y0_ref, lnw_ref, lnb_ref, fc1_ref, fc2_ref, b1_ref, b2_ref,
             w_ref, o_ref, xn_scr, *, ndd):
    dd = pl.program_id(1)

    @pl.when(dd == 0)
    def _():
        z = y0_ref[...]
        mu = jnp.mean(z, axis=1, keepdims=True)
        zc = z - mu
        var = jnp.mean(zc * zc, axis=1, keepdims=True)
        xn_scr[...] = (zc * lax.rsqrt(var + EPS) * lnw_ref[...]
                       + lnb_ref[...]).astype(BF16)

    h1 = lax.dot_general(xn_scr[...], fc1_ref[0].astype(BF16),
                         (((1,), (1,)), ((), ())),
                         preferred_element_type=F32) + b1_ref[0, 0]
    g = 0.5 * h1 * (1.0 + jnp.tanh(0.7978845608028654
                                   * (h1 + 0.044715 * h1 * h1 * h1)))
    part = lax.dot_general(g.astype(BF16), fc2_ref[0].astype(BF16),
                           (((1,), (1,)), ((), ())),
                           preferred_element_type=F32)

    @pl.when(dd == 0)
    def _():
        o_ref[...] = part

    @pl.when(dd != 0)
    def _():
        o_ref[...] += part

    @pl.when(dd == ndd - 1)
    def _():
        o_ref[...] = (o_ref[...] + b2_ref[0, 0]) * w_ref[0, 0][:, None]


def _k5(y0, lnw, lnb, fc1b, fc2b, b1s, b2s, wsel, *, kk, d, e, dd_pb):
    ddim = fc1b.shape[1]
    ndd = ddim // dd_pb
    return pl.pallas_call(
        functools.partial(_k5_body, ndd=ndd),
        grid=(e, ndd),
        in_specs=[
            pl.BlockSpec((kk, d), lambda ei, di: (ei, 0)),
            pl.BlockSpec((d,), lambda ei, di: (0,)),
            pl.BlockSpec((d,), lambda ei, di: (0,)),
            pl.BlockSpec((1, dd_pb, d), lambda ei, di: (ei, di, 0)),
            pl.BlockSpec((1, d, dd_pb), lambda ei, di: (ei, 0, di)),
            pl.BlockSpec((1, 1, dd_pb), lambda ei, di: (ei * ndd + di, 0, 0)),
            pl.BlockSpec((1, 1, d), lambda ei, di: (ei, 0, 0)),
            pl.BlockSpec((1, 1, kk), lambda ei, di: (ei, 0, 0)),
        ],
        out_specs=pl.BlockSpec((kk, d), lambda ei, di: (ei, 0)),
        out_shape=jax.ShapeDtypeStruct((kk * e, d), F32),
        scratch_shapes=[pltpu.VMEM((kk, d), BF16)],
        compiler_params=pltpu.CompilerParams(
            dimension_semantics=("arbitrary", "arbitrary")),
        interpret=_INTERPRET,
    )(y0, lnw, lnb, fc1b, fc2b,
      b1s.reshape(e * (ddim // dd_pb), 1, dd_pb), b2s.reshape(e, 1, d),
      wsel.reshape(e, 1, kk))


def _sca(probs_t, thr, ngt, *, kk):
    e, n = probs_t.shape
    mesh = plsc.VectorSubcoreMesh(core_axis_name="c", subcore_axis_name="s")

    @functools.partial(
        pl.kernel, mesh=mesh,
        compiler_params=pltpu.CompilerParams(needs_layout_passes=False),
        out_type=[jax.ShapeDtypeStruct((e, kk), jnp.int32),
                  jax.ShapeDtypeStruct((e, kk), F32)],
        scratch_types=[pltpu.VMEM((n,), F32),
                       pltpu.VMEM((16,), F32),
                       pltpu.VMEM((16,), jnp.int32),
                       pltpu.VMEM((kk + 16,), jnp.int32),
                       pltpu.VMEM((kk + 16,), F32)],
    )
    def k(pt_hbm, thr_hbm, ngt_hbm, idx_hbm, w_hbm, p_v, t_v, g_v, ib, wb):
        wid = lax.axis_index("s") * 2 + lax.axis_index("c")
        lanes = lax.iota(jnp.int32, 16)

        def dyng(vec, idx):
            return lax.gather(
                vec, idx[:, None],
                lax.GatherDimensionNumbers(offset_dims=(),
                                           collapsed_slice_dims=(0,),
                                           start_index_map=(0,)),
                (1,), mode=lax.GatherScatterMode.PROMISE_IN_BOUNDS)

        def bcast(vec, lane):
            return dyng(vec, jnp.zeros((16,), jnp.int32) + lane)

        def prefix16(x):
            for sh in (1, 2, 4, 8):
                shifted = dyng(x, jnp.maximum(lanes - sh, 0))
                x = x + jnp.where(lanes >= sh, shifted, 0)
            return x

        @pl.when(wid < e)
        def _():
            pltpu.sync_copy(pt_hbm.at[wid], p_v)
            pltpu.sync_copy(thr_hbm, t_v)
            pltpu.sync_copy(ngt_hbm, g_v)
            thr_v16 = bcast(t_v[...], wid)
            need_v16 = kk - bcast(g_v[...], wid)

            def body(c, carry):
                nt, ne = carry
                p = p_v[pl.ds(c * 16, 16)]
                gt = p > thr_v16
                eq = p == thr_v16
                eqp = prefix16(jnp.where(eq, 1, 0))
                take = gt | (eq & ((ne + eqp) <= need_v16))
                tkp = prefix16(jnp.where(take, 1, 0))
                pos = nt + tkp - 1
                plsc.store_scatter(ib, [pos], c * 16 + lanes, mask=take)
                plsc.store_scatter(wb, [pos], p, mask=take)
                nt = nt + bcast(tkp, 15)
                ne = ne + bcast(eqp, 15)
                return nt, ne

            z = jnp.zeros((16,), jnp.int32)
            lax.fori_loop(0, n // 16, body, (z, z))
            pltpu.sync_copy(ib.at[pl.ds(0, kk)], idx_hbm.at[wid])
            pltpu.sync_copy(wb.at[pl.ds(0, kk)], w_hbm.at[wid])

    return k(probs_t, thr, ngt)


def _scb(xf, idx_flat):
    n_tok, d = xf.shape
    rpw = n_tok // 32
    half = rpw // 2
    mesh = plsc.VectorSubcoreMesh(core_axis_name="c", subcore_axis_name="s")

    @functools.partial(
        pl.kernel, mesh=mesh,
        compiler_params=pltpu.CompilerParams(needs_layout_passes=False),
        out_type=jax.ShapeDtypeStruct((n_tok, d), F32),
        scratch_types=[pltpu.VMEM((2, half), jnp.int32),
                       pltpu.VMEM((half, d), F32),
                       pltpu.SemaphoreType.DMA],
    )
    def k(xf_hbm, idx_hbm, y0_hbm, ib, rows, sem):
        wid = lax.axis_index("s") * 2 + lax.axis_index("c")
        base = wid * rpw
        pltpu.sync_copy(idx_hbm.at[pl.ds(base, half)], ib.at[0])
        pltpu.sync_copy(idx_hbm.at[pl.ds(base + half, half)], ib.at[1])
        for j in range(2):
            pltpu.async_copy(xf_hbm.at[ib.at[j]], rows, sem).wait()
            pltpu.sync_copy(rows, y0_hbm.at[pl.ds(base + j * half, half)])

    return k(xf, idx_flat)


def _k6_body(xf_ref, idxf_ref, y_ref, o_ref, *, rows_pb):
    tok0 = pl.program_id(0) * rows_pb
    nslots = idxf_ref.shape[1]
    rows = tok0 + lax.broadcasted_iota(jnp.int32, (rows_pb, nslots), 0)
    p = jnp.where(idxf_ref[...] == rows, 1.0, 0.0).astype(BF16)
    delta = lax.dot_general(p, y_ref[...].astype(BF16), (((1,), (0,)), ((), ())),
                            preferred_element_type=F32)
    o_ref[...] = xf_ref[...] + delta


def _k6(xf, idx_flat, y, *, rows_pb):
    n_tok, d = xf.shape
    ns = idx_flat.shape[0]
    return pl.pallas_call(
        functools.partial(_k6_body, rows_pb=rows_pb),
        grid=(n_tok // rows_pb,),
        in_specs=[
            pl.BlockSpec((rows_pb, d), lambda i: (i, 0)),
            pl.BlockSpec((1, ns), lambda i: (0, 0)),
            pl.BlockSpec((ns, d), lambda i: (0, 0)),
        ],
        out_specs=pl.BlockSpec((rows_pb, d), lambda i: (i, 0)),
        out_shape=jax.ShapeDtypeStruct((n_tok, d), F32),
        interpret=_INTERPRET,
    )(xf, idx_flat.reshape(1, ns), y)


def kernel(x, t, ada_w, ada_b, ada_ln_w, ada_ln_b, Wqkv, qn_w, qn_b, kn_w, kn_b,
           Wout, mlp_ln_w, mlp_ln_b, Wg, fc1s, fc2s, b1s, b2s):
    b, s, d = x.shape
    e, ddim, _ = fc1s.shape
    dh = qn_w.shape[0]
    h = Wqkv.shape[0] // (3 * dh)
    n = b * s
    kk = n // e

    xf0 = x.reshape(n, d)
    shift = _k0(t.reshape(b, d), ada_w, ada_b)
    qkvb = _k1(xf0, shift, ada_ln_w, ada_ln_b, Wqkv,
               jnp.tile(qn_w, h), jnp.tile(qn_b, h),
               jnp.tile(kn_w, h), jnp.tile(kn_b, h),
               rows_pb=512, d=d, h=h, dh=dh)
    xf, probs_t = _k2(xf0, qkvb, Wout, Wg, b=b, s=s, d=d, h=h, dh=dh,
                      q_pb=256, e=e)
    thr, ngt = _k4(probs_t, kk=kk)

    idx, wsel = _sca(probs_t, thr, ngt, kk=kk)
    idx_flat = idx.reshape(-1)
    y0 = _scb(xf, idx_flat)

    y = _k5(y0, mlp_ln_w, mlp_ln_b, fc1s, fc2s, b1s, b2s, wsel,
            kk=kk, d=d, e=e, dd_pb=1024)

    out = _k6(xf, idx_flat, y, rows_pb=512)
    return out.reshape(b, s, d)
